# Initial kernel scaffold; baseline (speedup 1.0000x reference)
import jax
import jax.numpy as jnp
from jax.experimental import pallas as pl


def kernel(x, edge_index, W1, b1, W2, b2):
    raise NotImplementedError("write your pallas kernel here")


# speedup vs baseline: 14.7007x; 14.7007x over previous
import functools
import math

import jax
import jax.numpy as jnp
from jax import lax
from jax.experimental import pallas as pl
from jax.experimental.pallas import tpu as pltpu
from jax.experimental.pallas import tpu_sc as plsc

N_NODES = 10000
D_IN = 128
D_HID = 128
D2 = 16
NPAD = 10240
DUMMY = N_NODES
K = 128
NC = 2
NS = 16
NW = NC * NS
ROWS_PER_TILE = NPAD // NS
BM = 1024


def _sc_mesh():
    return plsc.VectorSubcoreMesh(core_axis_name="c", subcore_axis_name="s")


_SC_PARAMS = pltpu.CompilerParams(use_tc_tiling_on_sc=False)


def _make_deg_kernel(epad):
    chunks = epad // (NW * K)
    per_w = chunks * K

    @functools.partial(
        pl.kernel,
        out_type=[jax.ShapeDtypeStruct((NPAD, D2), jnp.float32),
                  jax.ShapeDtypeStruct((NPAD, D2), jnp.float32)],
        mesh=_sc_mesh(),
        compiler_params=_SC_PARAMS,
        scratch_types=[
            pltpu.VMEM((K,), jnp.int32),
            pltpu.VMEM((K, D2), jnp.float32),
            pltpu.VMEM_SHARED((NPAD, D2), jnp.float32),
        ],
    )
    def deg_kernel(dst_hbm, zeros_hbm, ones_hbm, out0, out1, didx, ones_v, shared):
        cid = lax.axis_index("c")
        sid = lax.axis_index("s")
        wid = sid * NC + cid
        base = wid * per_w
        r0 = sid * ROWS_PER_TILE

        pltpu.sync_copy(zeros_hbm.at[pl.ds(r0, ROWS_PER_TILE)],
                        shared.at[pl.ds(r0, ROWS_PER_TILE)])
        pltpu.sync_copy(ones_hbm, ones_v)
        plsc.subcore_barrier()

        @pl.loop(0, chunks)
        def _(j):
            pltpu.sync_copy(dst_hbm.at[pl.ds(base + j * K, K)], didx)
            pltpu.sync_copy(ones_v, shared.at[didx], add=True)

        plsc.subcore_barrier()

        @pl.when(cid == 0)
        def _():
            pltpu.sync_copy(shared.at[pl.ds(r0, ROWS_PER_TILE)],
                            out0.at[pl.ds(r0, ROWS_PER_TILE)])

        @pl.when(cid == 1)
        def _():
            pltpu.sync_copy(shared.at[pl.ds(r0, ROWS_PER_TILE)],
                            out1.at[pl.ds(r0, ROWS_PER_TILE)])

    return deg_kernel


def _make_agg_kernel(width, epad):
    chunks = epad // (NW * K)
    per_w = chunks * K

    @functools.partial(
        pl.kernel,
        out_type=[jax.ShapeDtypeStruct((NPAD, width), jnp.float32),
                  jax.ShapeDtypeStruct((NPAD, width), jnp.float32)],
        mesh=_sc_mesh(),
        compiler_params=_SC_PARAMS,
        scratch_types=[
            pltpu.VMEM((K,), jnp.int32),
            pltpu.VMEM((K,), jnp.int32),
            pltpu.VMEM((K, width), jnp.float32),
            pltpu.VMEM_SHARED((NPAD, width), jnp.float32),
            pltpu.SemaphoreType.DMA,
        ],
    )
    def agg_kernel(src_hbm, dst_hbm, tbl_hbm, out0, out1,
                   sidx, didx, rows, shared, sem):
        cid = lax.axis_index("c")
        sid = lax.axis_index("s")
        wid = sid * NC + cid
        base = wid * per_w
        r0 = sid * ROWS_PER_TILE

        pltpu.sync_copy(tbl_hbm.at[pl.ds(r0, ROWS_PER_TILE)],
                        shared.at[pl.ds(r0, ROWS_PER_TILE)])
        plsc.subcore_barrier()

        @pl.loop(0, chunks)
        def _(j):
            e0 = base + j * K
            pltpu.sync_copy(src_hbm.at[pl.ds(e0, K)], sidx)
            pltpu.sync_copy(dst_hbm.at[pl.ds(e0, K)], didx)
            pltpu.async_copy(tbl_hbm.at[sidx], rows, sem).wait()
            pltpu.sync_copy(rows, shared.at[didx], add=True)

        plsc.subcore_barrier()

        @pl.when(cid == 0)
        def _():
            pltpu.sync_copy(shared.at[pl.ds(r0, ROWS_PER_TILE)],
                            out0.at[pl.ds(r0, ROWS_PER_TILE)])

        @pl.when(cid == 1)
        def _():
            pltpu.sync_copy(shared.at[pl.ds(r0, ROWS_PER_TILE)],
                            out1.at[pl.ds(r0, ROWS_PER_TILE)])

    return agg_kernel


def _deg_g(d0, d1):
    return lax.rsqrt(d0[:, :1] + d1[:, :1] + 1.0)


def _mm1_body(x_ref, w_ref, d0_ref, d1_ref, o_ref):
    g = _deg_g(d0_ref[...], d1_ref[...])
    h = jnp.dot(x_ref[...], w_ref[...], preferred_element_type=jnp.float32)
    o_ref[...] = h * g


def _mid_body(p0_ref, p1_ref, h1_ref, d0_ref, d1_ref, b1_ref, w2_ref, o_ref):
    g = _deg_g(d0_ref[...], d1_ref[...])
    z = jnp.maximum(g * (p0_ref[...] + p1_ref[...] - h1_ref[...]) + b1_ref[...],
                    0.0)
    o_ref[...] = g * jnp.dot(z, w2_ref[...], preferred_element_type=jnp.float32)


def _fin_body(q0_ref, q1_ref, h2_ref, d0_ref, d1_ref, b2_ref, o_ref):
    g = _deg_g(d0_ref[...], d1_ref[...])
    o_ref[...] = g * (q0_ref[...] + q1_ref[...] - h2_ref[...]) + b2_ref[...]


def _row_spec(width):
    return pl.BlockSpec((BM, width), lambda i: (i, 0))


def _full_spec(shape):
    return pl.BlockSpec(shape, lambda i: (0,) * len(shape))


def kernel(x, edge_index, W1, b1, W2, b2):
    e = edge_index.astype(jnp.int32)
    n_edges = e.shape[1]
    epad = math.ceil(n_edges / (NW * K)) * (NW * K)
    pad = epad - n_edges
    src = jnp.concatenate([e[0], jnp.full((pad,), DUMMY, jnp.int32)])
    dst = jnp.concatenate([e[1], jnp.full((pad,), DUMMY, jnp.int32)])

    xp = jnp.zeros((NPAD, D_IN), jnp.float32).at[:x.shape[0]].set(x)
    w2p = jnp.zeros((D_HID, D2), jnp.float32).at[:, :W2.shape[1]].set(W2)
    b2p = jnp.zeros((1, D2), jnp.float32).at[0, :b2.shape[0]].set(b2)
    zeros16 = jnp.zeros((NPAD, D2), jnp.float32)
    ones16 = jnp.ones((K, D2), jnp.float32)

    grid = NPAD // BM

    d0, d1 = _make_deg_kernel(epad)(dst, zeros16, ones16)

    h1p = pl.pallas_call(
        _mm1_body,
        grid=(grid,),
        in_specs=[_row_spec(D_IN), _full_spec((D_IN, D_HID)),
                  _row_spec(D2), _row_spec(D2)],
        out_specs=_row_spec(D_HID),
        out_shape=jax.ShapeDtypeStruct((NPAD, D_HID), jnp.float32),
    )(xp, W1, d0, d1)

    p0, p1 = _make_agg_kernel(D_HID, epad)(src, dst, h1p)

    h2p = pl.pallas_call(
        _mid_body,
        grid=(grid,),
        in_specs=[_row_spec(D_HID), _row_spec(D_HID), _row_spec(D_HID),
                  _row_spec(D2), _row_spec(D2),
                  _full_spec((1, D_HID)), _full_spec((D_HID, D2))],
        out_specs=_row_spec(D2),
        out_shape=jax.ShapeDtypeStruct((NPAD, D2), jnp.float32),
    )(p0, p1, h1p, d0, d1, b1.reshape(1, D_HID), w2p)

    q0, q1 = _make_agg_kernel(D2, epad)(src, dst, h2p)

    outp = pl.pallas_call(
        _fin_body,
        grid=(grid,),
        in_specs=[_row_spec(D2), _row_spec(D2), _row_spec(D2),
                  _row_spec(D2), _row_spec(D2), _full_spec((1, D2))],
        out_specs=_row_spec(D2),
        out_shape=jax.ShapeDtypeStruct((NPAD, D2), jnp.float32),
    )(q0, q1, h2p, d0, d1, b2p)

    return outp[:x.shape[0], :W2.shape[1]]
